# baseline (speedup 1.0000x reference)
import functools

import jax
import jax.numpy as jnp
from jax import lax
from jax.experimental import pallas as pl
from jax.experimental.pallas import tpu as pltpu
from jax.experimental.pallas import tpu_sc as plsc

_BR = 2048
_BK = 256
_PASS_BLOCKS = (11, 11, 10)


def _pass_boundaries():
    ends, s = [], 0
    for nb in _PASS_BLOCKS:
        s += nb
        ends.append(s - 1)
    return tuple(ends)


_PASS_END = _pass_boundaries()
_PASS_START = (0,) + tuple(e + 1 for e in _PASS_END[:-1])


def _body(x_ref, e_ref, x2_ref, e2_ref, idx_ref, loss_ref,
          pv_ref, pi_ref, gv_ref, gi_ref, rv_ref):
    j = pl.program_id(1)
    x = x_ref[...]
    e = e_ref[...]
    x2 = x2_ref[...]
    e2 = e2_ref[...]
    xe = lax.dot_general(x, e, (((1,), (1,)), ((), ())),
                         preferred_element_type=jnp.float32)
    dist = jnp.sqrt(jnp.maximum((x2 + e2) - 2.0 * xe, 0.0))
    local_min = jnp.min(dist, axis=1, keepdims=True)
    cols = lax.broadcasted_iota(jnp.int32, dist.shape, 1)
    local_arg = jnp.min(
        jnp.where(dist == local_min, cols, jnp.int32(2**30)),
        axis=1, keepdims=True) + j * _BK

    is_pass_start = sum([j == s for s in _PASS_START]) > 0
    is_pass_end = sum([j == e for e in _PASS_END]) > 0

    @pl.when(is_pass_start)
    def _():
        pv_ref[...] = local_min
        pi_ref[...] = local_arg

    @pl.when(jnp.logical_not(is_pass_start))
    def _():
        better = local_min < pv_ref[...]
        pv_ref[...] = jnp.where(better, local_min, pv_ref[...])
        pi_ref[...] = jnp.where(better, local_arg, pi_ref[...])

    @pl.when(is_pass_end)
    def _():
        pv = pv_ref[...]
        pi = pi_ref[...]

        @pl.when(j == _PASS_END[0])
        def _():
            gv_ref[...] = pv
            gi_ref[...] = pi
            rv_ref[...] = pv

        @pl.when(j != _PASS_END[0])
        def _():
            acc_r = gv_ref[...].astype(jnp.bfloat16).astype(jnp.float32)
            win = pv < acc_r
            gv_ref[...] = jnp.where(win, pv, acc_r)
            gi_ref[...] = jnp.where(win, pi, gi_ref[...])
            rv_ref[...] = jnp.minimum(rv_ref[...], pv)

        @pl.when(j == _PASS_END[-1])
        def _():
            idx_ref[...] = gi_ref[...]
            i = pl.program_id(0)

            @pl.when(i == 0)
            def _():
                loss_ref[...] = jnp.zeros_like(loss_ref)

            rv = rv_ref[...]
            loss_ref[...] += jnp.sum(rv * rv)


def _dist_argmin(xf, ew, x2, e2):
    n, d = xf.shape
    k = ew.shape[0]
    grid = (n // _BR, k // _BK)
    idx, loss_sum = pl.pallas_call(
        _body,
        grid=grid,
        in_specs=[
            pl.BlockSpec((_BR, d), lambda i, j: (i, 0)),
            pl.BlockSpec((_BK, d), lambda i, j: (j, 0)),
            pl.BlockSpec((_BR, 1), lambda i, j: (i, 0)),
            pl.BlockSpec((1, _BK), lambda i, j: (0, j)),
        ],
        out_specs=[
            pl.BlockSpec((_BR, 1), lambda i, j: (i, 0)),
            pl.BlockSpec((1, 1), lambda i, j: (0, 0)),
        ],
        out_shape=[
            jax.ShapeDtypeStruct((n, 1), jnp.int32),
            jax.ShapeDtypeStruct((1, 1), jnp.float32),
        ],
        scratch_shapes=[
            pltpu.VMEM((_BR, 1), jnp.float32),
            pltpu.VMEM((_BR, 1), jnp.int32),
            pltpu.VMEM((_BR, 1), jnp.float32),
            pltpu.VMEM((_BR, 1), jnp.int32),
            pltpu.VMEM((_BR, 1), jnp.float32),
        ],
        compiler_params=pltpu.CompilerParams(
            dimension_semantics=("arbitrary", "arbitrary")),
    )(xf, ew, x2, e2)
    return idx.reshape(-1), loss_sum.reshape(())


@functools.lru_cache(maxsize=None)
def _make_gather(v, d, b):
    info = plsc.get_sparse_core_info()
    nw = info.num_cores * info.num_subcores
    assert b % (8 * nw) == 0 and d % info.num_lanes == 0
    b_per_w = b // nw
    ch = min(b_per_w, 256)
    assert b_per_w % ch == 0
    mesh = plsc.VectorSubcoreMesh(core_axis_name="c", subcore_axis_name="s")

    @functools.partial(
        pl.kernel, mesh=mesh,
        out_type=jax.ShapeDtypeStruct((b, d), jnp.float32),
        scratch_types=[
            pltpu.VMEM((ch,), jnp.int32),
            pltpu.VMEM((ch, d), jnp.float32),
            pltpu.SemaphoreType.DMA,
        ],
    )
    def gather(table_hbm, idx_hbm, out_hbm, idx_v, rows_v, sem):
        wid = lax.axis_index("s") * info.num_cores + lax.axis_index("c")
        base = wid * b_per_w
        for c in range(b_per_w // ch):
            off = base + c * ch
            pltpu.sync_copy(idx_hbm.at[pl.ds(off, ch)], idx_v)
            pltpu.async_copy(table_hbm.at[idx_v], rows_v, sem).wait()
            pltpu.sync_copy(rows_v, out_hbm.at[pl.ds(off, ch)])

    return gather


def kernel(x, embedding_weight):
    beta = 0.25
    bb, cc, dd = x.shape
    kk = embedding_weight.shape[0]
    xf = x.reshape(-1, dd)
    x2 = jnp.sum(x ** 2, axis=-1, keepdims=True).reshape(-1, 1)
    e2 = jnp.sum(embedding_weight ** 2, axis=-1).reshape(1, -1)
    idx, loss_sum = _dist_argmin(xf, embedding_weight, x2, e2)
    quant = _make_gather(kk, dd, bb * cc)(embedding_weight, idx)
    quantize_loss = loss_sum * ((1.0 + beta) / (bb * cc * dd))
    return (quant.reshape(bb, cc, dd), quantize_loss, idx.reshape(bb, cc))


# speedup vs baseline: 1.6199x; 1.6199x over previous
import functools

import jax
import jax.numpy as jnp
from jax import lax
from jax.experimental import pallas as pl
from jax.experimental.pallas import tpu as pltpu
from jax.experimental.pallas import tpu_sc as plsc

_BR = 2048
_BK = 256
_PASS_BLOCKS = (11, 11, 10)


def _pass_boundaries():
    ends, s = [], 0
    for nb in _PASS_BLOCKS:
        s += nb
        ends.append(s - 1)
    return tuple(ends)


_PASS_END = _pass_boundaries()
_PASS_START = (0,) + tuple(e + 1 for e in _PASS_END[:-1])


def _body(x_ref, e_ref, x2_ref, e2_ref, idx_ref, loss_ref,
          accv_ref, accj_ref, gv_ref, gi_ref, rv_ref):
    j = pl.program_id(1)
    x = x_ref[...]
    e = e_ref[...]
    x2 = x2_ref[...]
    e2 = e2_ref[...]
    xe = lax.dot_general(x, e, (((1,), (1,)), ((), ())),
                         preferred_element_type=jnp.float32)
    u = (x2 + e2) - 2.0 * xe

    is_pass_start = sum([j == s for s in _PASS_START]) > 0
    is_pass_end = sum([j == e for e in _PASS_END]) > 0

    @pl.when(is_pass_start)
    def _():
        accv_ref[...] = u
        accj_ref[...] = jnp.full(u.shape, j, jnp.int32)

    @pl.when(jnp.logical_not(is_pass_start))
    def _():
        better = u < accv_ref[...]
        accv_ref[...] = jnp.where(better, u, accv_ref[...])
        accj_ref[...] = jnp.where(better, j, accj_ref[...])

    @pl.when(is_pass_end)
    def _():
        s = jnp.sqrt(jnp.maximum(accv_ref[...], 0.0))
        pv = jnp.min(s, axis=1, keepdims=True)
        cols = lax.broadcasted_iota(jnp.int32, s.shape, 1)
        kmat = accj_ref[...] * _BK + cols
        pi = jnp.min(jnp.where(s == pv, kmat, jnp.int32(2**30)),
                     axis=1, keepdims=True)

        @pl.when(j == _PASS_END[0])
        def _():
            gv_ref[...] = pv
            gi_ref[...] = pi
            rv_ref[...] = pv

        @pl.when(j != _PASS_END[0])
        def _():
            acc_r = gv_ref[...].astype(jnp.bfloat16).astype(jnp.float32)
            win = pv < acc_r
            gv_ref[...] = jnp.where(win, pv, acc_r)
            gi_ref[...] = jnp.where(win, pi, gi_ref[...])
            rv_ref[...] = jnp.minimum(rv_ref[...], pv)

        @pl.when(j == _PASS_END[-1])
        def _():
            idx_ref[...] = gi_ref[...]
            i = pl.program_id(0)

            @pl.when(i == 0)
            def _():
                loss_ref[...] = jnp.zeros_like(loss_ref)

            rv = rv_ref[...]
            loss_ref[...] += jnp.sum(rv * rv)


def _dist_argmin(xf, ew, x2, e2):
    n, d = xf.shape
    k = ew.shape[0]
    grid = (n // _BR, k // _BK)
    idx, loss_sum = pl.pallas_call(
        _body,
        grid=grid,
        in_specs=[
            pl.BlockSpec((_BR, d), lambda i, j: (i, 0)),
            pl.BlockSpec((_BK, d), lambda i, j: (j, 0)),
            pl.BlockSpec((_BR, 1), lambda i, j: (i, 0)),
            pl.BlockSpec((1, _BK), lambda i, j: (0, j)),
        ],
        out_specs=[
            pl.BlockSpec((_BR, 1), lambda i, j: (i, 0)),
            pl.BlockSpec((1, 1), lambda i, j: (0, 0)),
        ],
        out_shape=[
            jax.ShapeDtypeStruct((n, 1), jnp.int32),
            jax.ShapeDtypeStruct((1, 1), jnp.float32),
        ],
        scratch_shapes=[
            pltpu.VMEM((_BR, _BK), jnp.float32),
            pltpu.VMEM((_BR, _BK), jnp.int32),
            pltpu.VMEM((_BR, 1), jnp.float32),
            pltpu.VMEM((_BR, 1), jnp.int32),
            pltpu.VMEM((_BR, 1), jnp.float32),
        ],
        compiler_params=pltpu.CompilerParams(
            dimension_semantics=("arbitrary", "arbitrary")),
    )(xf, ew, x2, e2)
    return idx.reshape(-1), loss_sum.reshape(())


@functools.lru_cache(maxsize=None)
def _make_gather(v, d, b):
    info = plsc.get_sparse_core_info()
    nw = info.num_cores * info.num_subcores
    assert b % (8 * nw) == 0 and d % info.num_lanes == 0
    b_per_w = b // nw
    ch = min(b_per_w, 256)
    assert b_per_w % ch == 0
    mesh = plsc.VectorSubcoreMesh(core_axis_name="c", subcore_axis_name="s")

    @functools.partial(
        pl.kernel, mesh=mesh,
        out_type=jax.ShapeDtypeStruct((b, d), jnp.float32),
        scratch_types=[
            pltpu.VMEM((ch,), jnp.int32),
            pltpu.VMEM((ch, d), jnp.float32),
            pltpu.SemaphoreType.DMA,
        ],
    )
    def gather(table_hbm, idx_hbm, out_hbm, idx_v, rows_v, sem):
        wid = lax.axis_index("s") * info.num_cores + lax.axis_index("c")
        base = wid * b_per_w
        for c in range(b_per_w // ch):
            off = base + c * ch
            pltpu.sync_copy(idx_hbm.at[pl.ds(off, ch)], idx_v)
            pltpu.async_copy(table_hbm.at[idx_v], rows_v, sem).wait()
            pltpu.sync_copy(rows_v, out_hbm.at[pl.ds(off, ch)])

    return gather


def kernel(x, embedding_weight):
    beta = 0.25
    bb, cc, dd = x.shape
    kk = embedding_weight.shape[0]
    xf = x.reshape(-1, dd)
    x2 = jnp.sum(x ** 2, axis=-1, keepdims=True).reshape(-1, 1)
    e2 = jnp.sum(embedding_weight ** 2, axis=-1).reshape(1, -1)
    idx, loss_sum = _dist_argmin(xf, embedding_weight, x2, e2)
    quant = _make_gather(kk, dd, bb * cc)(embedding_weight, idx)
    quantize_loss = loss_sum * ((1.0 + beta) / (bb * cc * dd))
    return (quant.reshape(bb, cc, dd), quantize_loss, idx.reshape(bb, cc))


# speedup vs baseline: 1.6241x; 1.0026x over previous
import functools

import jax
import jax.numpy as jnp
from jax import lax
from jax.experimental import pallas as pl
from jax.experimental.pallas import tpu as pltpu
from jax.experimental.pallas import tpu_sc as plsc

_BR = 2048
_BK = 256
_PASS_BLOCKS = (11, 11, 10)


def _pass_boundaries():
    ends, s = [], 0
    for nb in _PASS_BLOCKS:
        s += nb
        ends.append(s - 1)
    return tuple(ends)


_PASS_END = _pass_boundaries()
_PASS_START = (0,) + tuple(e + 1 for e in _PASS_END[:-1])


def _body(x_ref, e_ref, x2_ref, e2_ref, idx_ref, loss_ref,
          accv_ref, accj_ref, gv_ref, gi_ref, rv_ref):
    j = pl.program_id(1)
    x = x_ref[...]
    e = e_ref[...]
    x2 = x2_ref[...]
    e2 = e2_ref[...]
    xe2 = lax.dot_general(x, e + e, (((1,), (1,)), ((), ())),
                          preferred_element_type=jnp.float32)
    u = (x2 + e2) - xe2

    is_pass_start = sum([j == s for s in _PASS_START]) > 0
    is_pass_end = sum([j == e for e in _PASS_END]) > 0

    @pl.when(is_pass_start)
    def _():
        accv_ref[...] = u
        accj_ref[...] = jnp.full(u.shape, j, jnp.int32)

    @pl.when(jnp.logical_not(is_pass_start))
    def _():
        better = u < accv_ref[...]
        accv_ref[...] = jnp.where(better, u, accv_ref[...])
        accj_ref[...] = jnp.where(better, j, accj_ref[...])

    @pl.when(is_pass_end)
    def _():
        s = jnp.sqrt(jnp.maximum(accv_ref[...], 0.0))
        pv = jnp.min(s, axis=1, keepdims=True)
        cols = lax.broadcasted_iota(jnp.int32, s.shape, 1)
        kmat = accj_ref[...] * _BK + cols
        pi = jnp.min(jnp.where(s == pv, kmat, jnp.int32(2**30)),
                     axis=1, keepdims=True)

        @pl.when(j == _PASS_END[0])
        def _():
            gv_ref[...] = pv
            gi_ref[...] = pi
            rv_ref[...] = pv

        @pl.when(j != _PASS_END[0])
        def _():
            acc_r = gv_ref[...].astype(jnp.bfloat16).astype(jnp.float32)
            win = pv < acc_r
            gv_ref[...] = jnp.where(win, pv, acc_r)
            gi_ref[...] = jnp.where(win, pi, gi_ref[...])
            rv_ref[...] = jnp.minimum(rv_ref[...], pv)

        @pl.when(j == _PASS_END[-1])
        def _():
            idx_ref[...] = gi_ref[...]
            i = pl.program_id(0)

            @pl.when(i == 0)
            def _():
                loss_ref[...] = jnp.zeros_like(loss_ref)

            rv = rv_ref[...]
            loss_ref[...] += jnp.sum(rv * rv)


def _dist_argmin(xf, ew, x2, e2):
    n, d = xf.shape
    k = ew.shape[0]
    grid = (n // _BR, k // _BK)
    idx, loss_sum = pl.pallas_call(
        _body,
        grid=grid,
        in_specs=[
            pl.BlockSpec((_BR, d), lambda i, j: (i, 0)),
            pl.BlockSpec((_BK, d), lambda i, j: (j, 0)),
            pl.BlockSpec((_BR, 1), lambda i, j: (i, 0)),
            pl.BlockSpec((1, _BK), lambda i, j: (0, j)),
        ],
        out_specs=[
            pl.BlockSpec((_BR, 1), lambda i, j: (i, 0)),
            pl.BlockSpec((1, 1), lambda i, j: (0, 0)),
        ],
        out_shape=[
            jax.ShapeDtypeStruct((n, 1), jnp.int32),
            jax.ShapeDtypeStruct((1, 1), jnp.float32),
        ],
        scratch_shapes=[
            pltpu.VMEM((_BR, _BK), jnp.float32),
            pltpu.VMEM((_BR, _BK), jnp.int32),
            pltpu.VMEM((_BR, 1), jnp.float32),
            pltpu.VMEM((_BR, 1), jnp.int32),
            pltpu.VMEM((_BR, 1), jnp.float32),
        ],
        compiler_params=pltpu.CompilerParams(
            dimension_semantics=("arbitrary", "arbitrary")),
    )(xf, ew, x2, e2)
    return idx.reshape(-1), loss_sum.reshape(())


@functools.lru_cache(maxsize=None)
def _make_gather(v, d, b):
    info = plsc.get_sparse_core_info()
    nw = info.num_cores * info.num_subcores
    assert b % (8 * nw) == 0 and d % info.num_lanes == 0
    b_per_w = b // nw
    ch = min(b_per_w, 256)
    assert b_per_w % ch == 0
    mesh = plsc.VectorSubcoreMesh(core_axis_name="c", subcore_axis_name="s")

    @functools.partial(
        pl.kernel, mesh=mesh,
        out_type=jax.ShapeDtypeStruct((b, d), jnp.float32),
        scratch_types=[
            pltpu.VMEM((ch,), jnp.int32),
            pltpu.VMEM((ch, d), jnp.float32),
            pltpu.SemaphoreType.DMA,
        ],
    )
    def gather(table_hbm, idx_hbm, out_hbm, idx_v, rows_v, sem):
        wid = lax.axis_index("s") * info.num_cores + lax.axis_index("c")
        base = wid * b_per_w
        for c in range(b_per_w // ch):
            off = base + c * ch
            pltpu.sync_copy(idx_hbm.at[pl.ds(off, ch)], idx_v)
            pltpu.async_copy(table_hbm.at[idx_v], rows_v, sem).wait()
            pltpu.sync_copy(rows_v, out_hbm.at[pl.ds(off, ch)])

    return gather


def kernel(x, embedding_weight):
    beta = 0.25
    bb, cc, dd = x.shape
    kk = embedding_weight.shape[0]
    xf = x.reshape(-1, dd)
    x2 = jnp.sum(x ** 2, axis=-1, keepdims=True).reshape(-1, 1)
    e2 = jnp.sum(embedding_weight ** 2, axis=-1).reshape(1, -1)
    idx, loss_sum = _dist_argmin(xf, embedding_weight, x2, e2)
    quant = _make_gather(kk, dd, bb * cc)(embedding_weight, idx)
    quantize_loss = loss_sum * ((1.0 + beta) / (bb * cc * dd))
    return (quant.reshape(bb, cc, dd), quantize_loss, idx.reshape(bb, cc))
